# Initial kernel scaffold; baseline (speedup 1.0000x reference)
import jax
import jax.numpy as jnp
from jax.experimental import pallas as pl


def kernel(x, ln1_w, ln1_b, qkv_w, qkv_b, proj_w, proj_b, ln2_w, ln2_b, es1_w, es1_b, es2_w, es2_b, el1_w, el1_b, el2_w, el2_b, token_types):
    raise NotImplementedError("write your pallas kernel here")


# speedup vs baseline: 2.0556x; 2.0556x over previous
import jax
import jax.numpy as jnp
from jax.experimental import pallas as pl

B, N, C, H, HID = 2, 2048, 1024, 16, 4096
DH = C // H
SCALE = DH ** -0.5
BN = B * N
BM = 512
BQ = 512
NQT = N // BQ


def _ln(x, w, b, eps=1e-5):
    mu = jnp.mean(x, axis=-1, keepdims=True)
    xc = x - mu
    var = jnp.mean(xc * xc, axis=-1, keepdims=True)
    return xc * jax.lax.rsqrt(var + eps) * w + b


def _dot_t(a, w):
    return jax.lax.dot_general(a, w, (((1,), (1,)), ((), ())),
                               preferred_element_type=jnp.float32)


def _ln_qkv_kernel(x_ref, lnw_ref, lnb_ref, w_ref, b_ref, o_ref):
    h = _ln(x_ref[...], lnw_ref[...], lnb_ref[...])
    acc = _dot_t(h.astype(jnp.bfloat16), w_ref[...])
    o_ref[...] = (acc + b_ref[...]).astype(jnp.bfloat16)


def _attn_kernel(q_ref, kv_ref, o_ref):
    for h in range(H):
        q = q_ref[:, h * DH:(h + 1) * DH]
        k = kv_ref[:, C + h * DH:C + (h + 1) * DH]
        v = kv_ref[:, 2 * C + h * DH:2 * C + (h + 1) * DH]
        s = _dot_t(q, k) * SCALE
        m = jnp.max(s, axis=-1, keepdims=True)
        p = jnp.exp(s - m)
        p = p / jnp.sum(p, axis=-1, keepdims=True)
        o_ref[:, h * DH:(h + 1) * DH] = jnp.dot(
            p.astype(jnp.bfloat16), v,
            preferred_element_type=jnp.float32).astype(jnp.bfloat16)


def _proj_ln2_kernel(x_ref, a_ref, w_ref, b_ref, lnw_ref, lnb_ref,
                     x2_ref, h2_ref):
    a = _dot_t(a_ref[...], w_ref[...]) + b_ref[...]
    x2 = x_ref[...] + a
    x2_ref[...] = x2
    h2_ref[...] = _ln(x2, lnw_ref[...], lnb_ref[...]).astype(jnp.bfloat16)


def _moe_kernel(x2_ref, h2_ref, m_ref,
                s1_ref, s1b_ref, s2_ref, s2b_ref,
                l1_ref, l1b_ref, l2_ref, l2b_ref, o_ref):
    h2 = h2_ref[...]

    def expert(w1, b1, w2, b2):
        hh = _dot_t(h2, w1) + b1
        g = (0.5 * hh * (1.0 + jax.lax.erf(hh * (2.0 ** -0.5)))).astype(jnp.bfloat16)
        return _dot_t(g, w2) + b2

    out_s = expert(s1_ref[...], s1b_ref[...], s2_ref[...], s2b_ref[...])
    out_l = expert(l1_ref[...], l1b_ref[...], l2_ref[...], l2b_ref[...])
    sel = jnp.where(m_ref[...] > 0.5, out_l, out_s)
    o_ref[...] = x2_ref[...] + sel


def kernel(x, ln1_w, ln1_b, qkv_w, qkv_b, proj_w, proj_b, ln2_w, ln2_b,
           es1_w, es1_b, es2_w, es2_b, el1_w, el1_b, el2_w, el2_b,
           token_types):
    xf = x.reshape(BN, C)
    ln1_w2 = ln1_w.reshape(1, C)
    ln1_b2 = ln1_b.reshape(1, C)
    ln2_w2 = ln2_w.reshape(1, C)
    ln2_b2 = ln2_b.reshape(1, C)
    qkv_wb = qkv_w.astype(jnp.bfloat16)
    qkv_b2 = qkv_b.reshape(1, 3 * C)
    proj_wb = proj_w.astype(jnp.bfloat16)
    proj_b2 = proj_b.reshape(1, C)
    mask = (token_types.reshape(BN, 1) == 1).astype(jnp.float32)

    qkv = pl.pallas_call(
        _ln_qkv_kernel,
        grid=(BN // BM,),
        in_specs=[
            pl.BlockSpec((BM, C), lambda i: (i, 0)),
            pl.BlockSpec((1, C), lambda i: (0, 0)),
            pl.BlockSpec((1, C), lambda i: (0, 0)),
            pl.BlockSpec((3 * C, C), lambda i: (0, 0)),
            pl.BlockSpec((1, 3 * C), lambda i: (0, 0)),
        ],
        out_specs=pl.BlockSpec((BM, 3 * C), lambda i: (i, 0)),
        out_shape=jax.ShapeDtypeStruct((BN, 3 * C), jnp.bfloat16),
    )(xf, ln1_w2, ln1_b2, qkv_wb, qkv_b2)

    a = pl.pallas_call(
        _attn_kernel,
        grid=(B, NQT),
        in_specs=[
            pl.BlockSpec((BQ, 3 * C), lambda b, i: (b * NQT + i, 0)),
            pl.BlockSpec((N, 3 * C), lambda b, i: (b, 0)),
        ],
        out_specs=pl.BlockSpec((BQ, C), lambda b, i: (b * NQT + i, 0)),
        out_shape=jax.ShapeDtypeStruct((BN, C), jnp.bfloat16),
    )(qkv, qkv)

    x2, h2 = pl.pallas_call(
        _proj_ln2_kernel,
        grid=(BN // BM,),
        in_specs=[
            pl.BlockSpec((BM, C), lambda i: (i, 0)),
            pl.BlockSpec((BM, C), lambda i: (i, 0)),
            pl.BlockSpec((C, C), lambda i: (0, 0)),
            pl.BlockSpec((1, C), lambda i: (0, 0)),
            pl.BlockSpec((1, C), lambda i: (0, 0)),
            pl.BlockSpec((1, C), lambda i: (0, 0)),
        ],
        out_specs=[
            pl.BlockSpec((BM, C), lambda i: (i, 0)),
            pl.BlockSpec((BM, C), lambda i: (i, 0)),
        ],
        out_shape=[
            jax.ShapeDtypeStruct((BN, C), jnp.float32),
            jax.ShapeDtypeStruct((BN, C), jnp.bfloat16),
        ],
    )(xf, a, proj_wb, proj_b2, ln2_w2, ln2_b2)

    out = pl.pallas_call(
        _moe_kernel,
        grid=(BN // BM,),
        in_specs=[
            pl.BlockSpec((BM, C), lambda i: (i, 0)),
            pl.BlockSpec((BM, C), lambda i: (i, 0)),
            pl.BlockSpec((BM, 1), lambda i: (i, 0)),
            pl.BlockSpec((HID, C), lambda i: (0, 0)),
            pl.BlockSpec((1, HID), lambda i: (0, 0)),
            pl.BlockSpec((C, HID), lambda i: (0, 0)),
            pl.BlockSpec((1, C), lambda i: (0, 0)),
            pl.BlockSpec((HID, C), lambda i: (0, 0)),
            pl.BlockSpec((1, HID), lambda i: (0, 0)),
            pl.BlockSpec((C, HID), lambda i: (0, 0)),
            pl.BlockSpec((1, C), lambda i: (0, 0)),
        ],
        out_specs=pl.BlockSpec((BM, C), lambda i: (i, 0)),
        out_shape=jax.ShapeDtypeStruct((BN, C), jnp.float32),
    )(x2, h2, mask,
      es1_w.astype(jnp.bfloat16), es1_b.reshape(1, HID),
      es2_w.astype(jnp.bfloat16), es2_b.reshape(1, C),
      el1_w.astype(jnp.bfloat16), el1_b.reshape(1, HID),
      el2_w.astype(jnp.bfloat16), el2_b.reshape(1, C))

    return out.reshape(B, N, C)
